# baseline (speedup 1.0000x reference)
import jax
import jax.numpy as jnp
from jax.experimental import pallas as pl
from jax.experimental.pallas import tpu as pltpu

_N = 100000
_D = 128
_KEEP = 25000
_BLK = 2048


def _scores_body(x_ref, o_ref):
    t = x_ref[...]
    t = t * t
    u = t[:, 0:8]
    for k in range(1, 16):
        u = u + t[:, 8 * k:8 * k + 8]
    v = u[:, :4] + u[:, 4:]
    w = v[:, :2] + v[:, 2:]
    o_ref[...] = jnp.sqrt(w[:, 0] + w[:, 1])


def _scores(x):
    return pl.pallas_call(
        _scores_body,
        grid=(pl.cdiv(_N, _BLK),),
        in_specs=[pl.BlockSpec((_BLK, _D), lambda i: (i, 0))],
        out_specs=pl.BlockSpec((_BLK,), lambda i: (i,)),
        out_shape=jax.ShapeDtypeStruct((_N,), jnp.float32),
    )(x)


def kernel(x, pos, batch, W, b):
    scores = _scores(x)
    perm = jnp.argsort(scores)[-_KEEP:]
    x_c = x[perm] @ W + b
    pos_c = pos[perm]
    batch_c = batch[perm]
    return (x_c, pos_c, batch_c)


# speedup vs baseline: 1.4452x; 1.4452x over previous
import jax
import jax.numpy as jnp
from jax import lax
from jax.experimental import pallas as pl
from jax.experimental.pallas import tpu as pltpu
from jax.experimental.pallas import tpu_sc as plsc

_N = 100000
_D = 128
_KEEP = 25000
_START = _N - _KEEP

_NW = 16
_SHARD = 6256
_SHARD_LAST = _N - 15 * _SHARD
_NV = _SHARD // 16
_NV_LAST = _SHARD_LAST // 16

_OSH = 1568
_OSH_LAST = _KEEP - 15 * _OSH
_XCH = 392


_BLK = 2048


def _scores_body(x_ref, o_ref):
    t = x_ref[...]
    t = t * t
    u = t[:, 0:8]
    for k in range(1, 16):
        u = u + t[:, 8 * k:8 * k + 8]
    v = u[:, :4] + u[:, 4:]
    w = v[:, :2] + v[:, 2:]
    o_ref[...] = jnp.sqrt(w[:, 0] + w[:, 1])


def _scores(x):
    return pl.pallas_call(
        _scores_body,
        grid=(pl.cdiv(_N, _BLK),),
        in_specs=[pl.BlockSpec((_BLK, _D), lambda i: (i, 0))],
        out_specs=pl.BlockSpec((_BLK,), lambda i: (i,)),
        out_shape=jax.ShapeDtypeStruct((_N,), jnp.float32),
    )(x)


def _sc_body(keys_hbm, p0_hbm, p1_hbm, p2_hbm, batch_hbm, x_hbm,
             perm_hbm, p0c_hbm, p1c_hbm, p2c_hbm, batchc_hbm, xsel_hbm,
             kA, vA, kB, vB, histg,
             key_loc, val_loc, pos_loc, grid_loc, off_loc, hist_loc,
             idx_loc, xtmp, ptmp, btmp, sem, sem2):
    w = lax.axis_index("s")
    is_last = w == _NW - 1
    not_last = jnp.logical_not(is_last)
    base = w * _SHARD
    nv = jnp.where(is_last, _NV_LAST, _NV)
    lanes = lax.iota(jnp.int32, 16)

    def zero_hist():
        for i in range(16):
            hist_loc[pl.ds(16 * i, 16)] = jnp.zeros((16,), jnp.int32)

    def compute_offsets():
        pltpu.sync_copy(histg, grid_loc)
        carry = jnp.int32(0)
        for i in range(16):
            tot = jnp.zeros((16,), jnp.int32)
            for ww in range(_NW):
                tot = tot + grid_loc[ww, pl.ds(16 * i, 16)]
            csum = plsc.cumsum(tot)
            excl = (csum - tot) + carry
            carry = carry + jnp.sum(tot)
            part = jnp.zeros((16,), jnp.int32)
            for ww in range(_NW - 1):
                row = grid_loc[ww, pl.ds(16 * i, 16)]
                part = part + row * (w > ww).astype(jnp.int32)
            off_loc[pl.ds(16 * i, 16)] = excl + part

    def radix_pass(shift, first, kout, vout):
        zero_hist()

        def hist_body(i, c):
            if first:
                val_loc[pl.ds(16 * i, 16)] = base + 16 * i + lanes
            kv = key_loc[pl.ds(16 * i, 16)]
            d = lax.shift_right_logical(kv, shift) & 255
            cnt, last = plsc.scan_count(d)
            plsc.addupdate_scatter(hist_loc, [d], cnt, mask=last)
            return c

        lax.fori_loop(0, nv, hist_body, jnp.int32(0))
        pltpu.sync_copy(hist_loc, histg.at[w])
        plsc.subcore_barrier()
        compute_offsets()

        def perm_body(i, c):
            kv = key_loc[pl.ds(16 * i, 16)]
            d = lax.shift_right_logical(kv, shift) & 255
            cnt, last = plsc.scan_count(d)
            b_ = plsc.load_gather(off_loc, [d])
            pos_loc[pl.ds(16 * i, 16)] = b_ + cnt - 1
            plsc.addupdate_scatter(off_loc, [d], cnt, mask=last)
            return c

        lax.fori_loop(0, nv, perm_body, jnp.int32(0))

        def scat(sz):
            pltpu.async_copy(key_loc.at[pl.ds(0, sz)],
                             kout.at[pos_loc.at[pl.ds(0, sz)]], sem).wait()
            pltpu.async_copy(val_loc.at[pl.ds(0, sz)],
                             vout.at[pos_loc.at[pl.ds(0, sz)]], sem).wait()

        @pl.when(not_last)
        def _():
            scat(_SHARD)

        @pl.when(is_last)
        def _():
            scat(_SHARD_LAST)

        plsc.subcore_barrier()

    def load_ping(kin, vin, with_vals):
        def ld(sz):
            pltpu.sync_copy(kin.at[pl.ds(base, sz)],
                            key_loc.at[pl.ds(0, sz)])
            if with_vals:
                pltpu.sync_copy(vin.at[pl.ds(base, sz)],
                                val_loc.at[pl.ds(0, sz)])

        @pl.when(not_last)
        def _():
            ld(_SHARD)

        @pl.when(is_last)
        def _():
            ld(_SHARD_LAST)

    load_ping(keys_hbm, None, False)
    radix_pass(0, True, kB, vB)
    load_ping(kB, vB, True)
    radix_pass(8, False, kA, vA)
    load_ping(kA, vA, True)
    radix_pass(16, False, kB, vB)
    load_ping(kB, vB, True)
    radix_pass(24, False, kA, vA)

    def out_phase(sz, nxch, xrem):
        obase = w * _OSH
        pltpu.sync_copy(vA.at[pl.ds(_START + obase, sz)],
                        idx_loc.at[pl.ds(0, sz)])
        pltpu.sync_copy(idx_loc.at[pl.ds(0, sz)],
                        perm_hbm.at[pl.ds(obase, sz)])
        pltpu.async_copy(batch_hbm.at[idx_loc.at[pl.ds(0, sz)]],
                         btmp.at[pl.ds(0, sz)], sem).wait()
        pltpu.sync_copy(btmp.at[pl.ds(0, sz)],
                        batchc_hbm.at[pl.ds(obase, sz)])
        for src, dst in ((p0_hbm, p0c_hbm), (p1_hbm, p1c_hbm),
                         (p2_hbm, p2c_hbm)):
            pltpu.async_copy(src.at[idx_loc.at[pl.ds(0, sz)]],
                             ptmp.at[pl.ds(0, sz)], sem).wait()
            pltpu.sync_copy(ptmp.at[pl.ds(0, sz)],
                            dst.at[pl.ds(obase, sz)])
        for j in range(nxch):
            pltpu.async_copy(x_hbm.at[idx_loc.at[pl.ds(j * _XCH, _XCH)]],
                             xtmp, sem2).wait()
            pltpu.sync_copy(xtmp,
                            xsel_hbm.at[pl.ds(obase + j * _XCH, _XCH)])
        if xrem:
            pltpu.async_copy(x_hbm.at[idx_loc.at[pl.ds(nxch * _XCH, xrem)]],
                             xtmp.at[pl.ds(0, xrem)], sem2).wait()
            pltpu.sync_copy(xtmp.at[pl.ds(0, xrem)],
                            xsel_hbm.at[pl.ds(obase + nxch * _XCH, xrem)])

    @pl.when(not_last)
    def _():
        out_phase(_OSH, _OSH // _XCH, 0)

    @pl.when(is_last)
    def _():
        out_phase(_OSH_LAST, _OSH_LAST // _XCH, _OSH_LAST % _XCH)


def _sc_sort_gather(keys, p0, p1, p2, batch, x):
    mesh = plsc.VectorSubcoreMesh(
        core_axis_name="c", subcore_axis_name="s", num_cores=1)
    f = pl.kernel(
        _sc_body,
        out_type=[
            jax.ShapeDtypeStruct((_KEEP,), jnp.int32),
            jax.ShapeDtypeStruct((_KEEP,), jnp.float32),
            jax.ShapeDtypeStruct((_KEEP,), jnp.float32),
            jax.ShapeDtypeStruct((_KEEP,), jnp.float32),
            jax.ShapeDtypeStruct((_KEEP,), jnp.int32),
            jax.ShapeDtypeStruct((_KEEP, _D), jnp.float32),
        ],
        mesh=mesh,
        compiler_params=pltpu.CompilerParams(needs_layout_passes=False),
        scratch_types=[
            pltpu.VMEM_SHARED((_N + 96,), jnp.int32),
            pltpu.VMEM_SHARED((_N + 96,), jnp.int32),
            pltpu.VMEM_SHARED((_N + 96,), jnp.int32),
            pltpu.VMEM_SHARED((_N + 96,), jnp.int32),
            pltpu.VMEM_SHARED((_NW, 256), jnp.int32),
            pltpu.VMEM((_SHARD,), jnp.int32),
            pltpu.VMEM((_SHARD,), jnp.int32),
            pltpu.VMEM((_SHARD,), jnp.int32),
            pltpu.VMEM((_NW, 256), jnp.int32),
            pltpu.VMEM((256,), jnp.int32),
            pltpu.VMEM((256,), jnp.int32),
            pltpu.VMEM((_OSH,), jnp.int32),
            pltpu.VMEM((_XCH, _D), jnp.float32),
            pltpu.VMEM((_OSH,), jnp.float32),
            pltpu.VMEM((_OSH,), jnp.int32),
            pltpu.SemaphoreType.DMA,
            pltpu.SemaphoreType.DMA,
        ],
    )
    return f(keys, p0, p1, p2, batch, x)


_MBLK = 2048


def _mm_body(xs_ref, w_ref, b_ref, o_ref):
    o_ref[...] = (
        jnp.dot(xs_ref[...], w_ref[...], preferred_element_type=jnp.float32)
        + b_ref[...]
    )


def _matmul(x_sel, W, b):
    return pl.pallas_call(
        _mm_body,
        grid=(pl.cdiv(_KEEP, _MBLK),),
        in_specs=[
            pl.BlockSpec((_MBLK, _D), lambda i: (i, 0)),
            pl.BlockSpec((_D, _D), lambda i: (0, 0)),
            pl.BlockSpec((1, _D), lambda i: (0, 0)),
        ],
        out_specs=pl.BlockSpec((_MBLK, _D), lambda i: (i, 0)),
        out_shape=jax.ShapeDtypeStruct((_KEEP, _D), jnp.float32),
    )(x_sel, W, b.reshape(1, _D))


def kernel(x, pos, batch, W, b):
    scores = _scores(x)
    keys = lax.bitcast_convert_type(scores, jnp.int32)
    perm, p0c, p1c, p2c, batch_c, x_sel = _sc_sort_gather(
        keys, pos[:, 0], pos[:, 1], pos[:, 2], batch, x)
    x_c = _matmul(x_sel, W, b)
    pos_c = jnp.stack([p0c, p1c, p2c], axis=1)
    return (x_c, pos_c, batch_c)


# speedup vs baseline: 3.6300x; 2.5117x over previous
import jax
import jax.numpy as jnp
from jax import lax
from jax.experimental import pallas as pl
from jax.experimental.pallas import tpu as pltpu
from jax.experimental.pallas import tpu_sc as plsc

_N = 100000
_D = 128
_KEEP = 25000
_START = _N - _KEEP

_NW = 16
_SHARD = 6256
_SHARD_LAST = _N - 15 * _SHARD
_NV = _SHARD // 16
_NV_LAST = _SHARD_LAST // 16

_OSH = 1568
_OSH_LAST = _KEEP - 15 * _OSH
_XCH = 392


_BLK = 2048


def _scores_body(x_ref, o_ref):
    t = x_ref[...]
    t = (t * t).T
    u = t[0:8, :]
    for k in range(1, 16):
        u = u + t[8 * k:8 * k + 8, :]
    v = u[0:4, :] + u[4:8, :]
    w = v[0:2, :] + v[2:4, :]
    o_ref[...] = jnp.sqrt(w[0, :] + w[1, :])


def _scores(x):
    return pl.pallas_call(
        _scores_body,
        grid=(pl.cdiv(_N, _BLK),),
        in_specs=[pl.BlockSpec((_BLK, _D), lambda i: (i, 0))],
        out_specs=pl.BlockSpec((_BLK,), lambda i: (i,)),
        out_shape=jax.ShapeDtypeStruct((_N,), jnp.float32),
    )(x)


def _sc_body(keys_hbm, p0_hbm, p1_hbm, p2_hbm, batch_hbm, x_hbm,
             perm_hbm, p0c_hbm, p1c_hbm, p2c_hbm, batchc_hbm, xsel_hbm,
             kA, vA, kB, vB, histg,
             key_loc, val_loc, pos_loc, grid_loc, off_loc, hist_loc,
             idx_loc, xtmp, ptmp, btmp, sem, sem2):
    w = lax.axis_index("s")
    is_last = w == _NW - 1
    not_last = jnp.logical_not(is_last)
    base = w * _SHARD
    nv = jnp.where(is_last, _NV_LAST, _NV)
    lanes = lax.iota(jnp.int32, 16)

    def zero_hist():
        for i in range(16):
            hist_loc[pl.ds(16 * i, 16)] = jnp.zeros((16,), jnp.int32)

    def compute_offsets():
        pltpu.sync_copy(histg, grid_loc)
        carry = jnp.int32(0)
        for i in range(16):
            tot = jnp.zeros((16,), jnp.int32)
            for ww in range(_NW):
                tot = tot + grid_loc[ww, pl.ds(16 * i, 16)]
            csum = plsc.cumsum(tot)
            excl = (csum - tot) + carry
            carry = carry + jnp.sum(tot)
            part = jnp.zeros((16,), jnp.int32)
            for ww in range(_NW - 1):
                row = grid_loc[ww, pl.ds(16 * i, 16)]
                part = part + row * (w > ww).astype(jnp.int32)
            off_loc[pl.ds(16 * i, 16)] = excl + part

    def radix_pass(shift, first, kout, vout):
        zero_hist()

        def hist_body(i, c):
            if first:
                val_loc[pl.ds(16 * i, 16)] = base + 16 * i + lanes
            kv = key_loc[pl.ds(16 * i, 16)]
            d = lax.shift_right_logical(kv, shift) & 255
            cnt, last = plsc.scan_count(d)
            plsc.addupdate_scatter(hist_loc, [d], cnt, mask=last)
            return c

        lax.fori_loop(0, nv, hist_body, jnp.int32(0))
        pltpu.sync_copy(hist_loc, histg.at[w])
        plsc.subcore_barrier()
        compute_offsets()

        def perm_body(i, c):
            kv = key_loc[pl.ds(16 * i, 16)]
            d = lax.shift_right_logical(kv, shift) & 255
            cnt, last = plsc.scan_count(d)
            b_ = plsc.load_gather(off_loc, [d])
            pos_loc[pl.ds(16 * i, 16)] = b_ + cnt - 1
            plsc.addupdate_scatter(off_loc, [d], cnt, mask=last)
            return c

        lax.fori_loop(0, nv, perm_body, jnp.int32(0))

        def scat(sz):
            pltpu.async_copy(key_loc.at[pl.ds(0, sz)],
                             kout.at[pos_loc.at[pl.ds(0, sz)]], sem).wait()
            pltpu.async_copy(val_loc.at[pl.ds(0, sz)],
                             vout.at[pos_loc.at[pl.ds(0, sz)]], sem).wait()

        @pl.when(not_last)
        def _():
            scat(_SHARD)

        @pl.when(is_last)
        def _():
            scat(_SHARD_LAST)

        plsc.subcore_barrier()

    def load_ping(kin, vin, with_vals):
        def ld(sz):
            pltpu.sync_copy(kin.at[pl.ds(base, sz)],
                            key_loc.at[pl.ds(0, sz)])
            if with_vals:
                pltpu.sync_copy(vin.at[pl.ds(base, sz)],
                                val_loc.at[pl.ds(0, sz)])

        @pl.when(not_last)
        def _():
            ld(_SHARD)

        @pl.when(is_last)
        def _():
            ld(_SHARD_LAST)

    load_ping(keys_hbm, None, False)
    radix_pass(0, True, kB, vB)
    load_ping(kB, vB, True)
    radix_pass(8, False, kA, vA)
    load_ping(kA, vA, True)
    radix_pass(16, False, kB, vB)
    load_ping(kB, vB, True)
    radix_pass(24, False, kA, vA)

    def out_phase(sz, nxch, xrem):
        obase = w * _OSH
        pltpu.sync_copy(vA.at[pl.ds(_START + obase, sz)],
                        idx_loc.at[pl.ds(0, sz)])
        pltpu.sync_copy(idx_loc.at[pl.ds(0, sz)],
                        perm_hbm.at[pl.ds(obase, sz)])
        pltpu.async_copy(batch_hbm.at[idx_loc.at[pl.ds(0, sz)]],
                         btmp.at[pl.ds(0, sz)], sem).wait()
        pltpu.sync_copy(btmp.at[pl.ds(0, sz)],
                        batchc_hbm.at[pl.ds(obase, sz)])
        for src, dst in ((p0_hbm, p0c_hbm), (p1_hbm, p1c_hbm),
                         (p2_hbm, p2c_hbm)):
            pltpu.async_copy(src.at[idx_loc.at[pl.ds(0, sz)]],
                             ptmp.at[pl.ds(0, sz)], sem).wait()
            pltpu.sync_copy(ptmp.at[pl.ds(0, sz)],
                            dst.at[pl.ds(obase, sz)])
        for j in range(nxch):
            pltpu.async_copy(x_hbm.at[idx_loc.at[pl.ds(j * _XCH, _XCH)]],
                             xtmp, sem2).wait()
            pltpu.sync_copy(xtmp,
                            xsel_hbm.at[pl.ds(obase + j * _XCH, _XCH)])
        if xrem:
            pltpu.async_copy(x_hbm.at[idx_loc.at[pl.ds(nxch * _XCH, xrem)]],
                             xtmp.at[pl.ds(0, xrem)], sem2).wait()
            pltpu.sync_copy(xtmp.at[pl.ds(0, xrem)],
                            xsel_hbm.at[pl.ds(obase + nxch * _XCH, xrem)])

    @pl.when(not_last)
    def _():
        out_phase(_OSH, _OSH // _XCH, 0)

    @pl.when(is_last)
    def _():
        out_phase(_OSH_LAST, _OSH_LAST // _XCH, _OSH_LAST % _XCH)


def _sc_sort_gather(keys, p0, p1, p2, batch, x):
    mesh = plsc.VectorSubcoreMesh(
        core_axis_name="c", subcore_axis_name="s", num_cores=1)
    f = pl.kernel(
        _sc_body,
        out_type=[
            jax.ShapeDtypeStruct((_KEEP,), jnp.int32),
            jax.ShapeDtypeStruct((_KEEP,), jnp.float32),
            jax.ShapeDtypeStruct((_KEEP,), jnp.float32),
            jax.ShapeDtypeStruct((_KEEP,), jnp.float32),
            jax.ShapeDtypeStruct((_KEEP,), jnp.int32),
            jax.ShapeDtypeStruct((_KEEP, _D), jnp.float32),
        ],
        mesh=mesh,
        compiler_params=pltpu.CompilerParams(needs_layout_passes=False),
        scratch_types=[
            pltpu.VMEM_SHARED((_N + 96,), jnp.int32),
            pltpu.VMEM_SHARED((_N + 96,), jnp.int32),
            pltpu.VMEM_SHARED((_N + 96,), jnp.int32),
            pltpu.VMEM_SHARED((_N + 96,), jnp.int32),
            pltpu.VMEM_SHARED((_NW, 256), jnp.int32),
            pltpu.VMEM((_SHARD,), jnp.int32),
            pltpu.VMEM((_SHARD,), jnp.int32),
            pltpu.VMEM((_SHARD,), jnp.int32),
            pltpu.VMEM((_NW, 256), jnp.int32),
            pltpu.VMEM((256,), jnp.int32),
            pltpu.VMEM((256,), jnp.int32),
            pltpu.VMEM((_OSH,), jnp.int32),
            pltpu.VMEM((_XCH, _D), jnp.float32),
            pltpu.VMEM((_OSH,), jnp.float32),
            pltpu.VMEM((_OSH,), jnp.int32),
            pltpu.SemaphoreType.DMA,
            pltpu.SemaphoreType.DMA,
        ],
    )
    return f(keys, p0, p1, p2, batch, x)


_MBLK = 2048


def _mm_body(xs_ref, w_ref, b_ref, o_ref):
    o_ref[...] = (
        jnp.dot(xs_ref[...], w_ref[...], preferred_element_type=jnp.float32)
        + b_ref[...]
    )


def _matmul(x_sel, W, b):
    return pl.pallas_call(
        _mm_body,
        grid=(pl.cdiv(_KEEP, _MBLK),),
        in_specs=[
            pl.BlockSpec((_MBLK, _D), lambda i: (i, 0)),
            pl.BlockSpec((_D, _D), lambda i: (0, 0)),
            pl.BlockSpec((1, _D), lambda i: (0, 0)),
        ],
        out_specs=pl.BlockSpec((_MBLK, _D), lambda i: (i, 0)),
        out_shape=jax.ShapeDtypeStruct((_KEEP, _D), jnp.float32),
    )(x_sel, W, b.reshape(1, _D))


def kernel(x, pos, batch, W, b):
    scores = _scores(x)
    keys = lax.bitcast_convert_type(scores, jnp.int32)
    perm, p0c, p1c, p2c, batch_c, x_sel = _sc_sort_gather(
        keys, pos[:, 0], pos[:, 1], pos[:, 2], batch, x)
    x_c = _matmul(x_sel, W, b)
    pos_c = jnp.stack([p0c, p1c, p2c], axis=1)
    return (x_c, pos_c, batch_c)


# speedup vs baseline: 3.6854x; 1.0153x over previous
import jax
import jax.numpy as jnp
from jax import lax
from jax.experimental import pallas as pl
from jax.experimental.pallas import tpu as pltpu
from jax.experimental.pallas import tpu_sc as plsc

_N = 100000
_D = 128
_KEEP = 25000
_START = _N - _KEEP

_NW = 16
_SHARD = 6256
_SHARD_LAST = _N - 15 * _SHARD
_NV = _SHARD // 16
_NV_LAST = _SHARD_LAST // 16

_OSH = 1568
_OSH_LAST = _KEEP - 15 * _OSH
_XCH = 392


_BLK = 2048


def _scores_body(x_ref, o_ref):
    t = x_ref[...]
    t = (t * t).T
    u = t[0:8, :]
    for k in range(1, 16):
        u = u + t[8 * k:8 * k + 8, :]
    v = u[0:4, :] + u[4:8, :]
    w = v[0:2, :] + v[2:4, :]
    o_ref[...] = jnp.sqrt(w[0, :] + w[1, :])


def _scores(x):
    return pl.pallas_call(
        _scores_body,
        grid=(pl.cdiv(_N, _BLK),),
        in_specs=[pl.BlockSpec((_BLK, _D), lambda i: (i, 0))],
        out_specs=pl.BlockSpec((_BLK,), lambda i: (i,)),
        out_shape=jax.ShapeDtypeStruct((_N,), jnp.float32),
    )(x)


def _sc_body(keys_hbm, p0_hbm, p1_hbm, p2_hbm, batch_hbm, x_hbm,
             perm_hbm, p0c_hbm, p1c_hbm, p2c_hbm, batchc_hbm, xsel_hbm,
             kA, vA, kB, vB, histg,
             key_loc, val_loc, pos_loc, grid_loc, off_loc, hist_loc,
             idx_loc, xtmp, ptmp, btmp, sem, sem2):
    w = lax.axis_index("s")
    is_last = w == _NW - 1
    not_last = jnp.logical_not(is_last)
    base = w * _SHARD
    nv = jnp.where(is_last, _NV_LAST, _NV)
    lanes = lax.iota(jnp.int32, 16)

    def zero_hist():
        for i in range(16):
            hist_loc[pl.ds(16 * i, 16)] = jnp.zeros((16,), jnp.int32)

    def compute_offsets():
        pltpu.sync_copy(histg, grid_loc)
        carry = jnp.int32(0)
        for i in range(16):
            tot = jnp.zeros((16,), jnp.int32)
            for ww in range(_NW):
                tot = tot + grid_loc[ww, pl.ds(16 * i, 16)]
            csum = plsc.cumsum(tot)
            excl = (csum - tot) + carry
            carry = carry + jnp.sum(tot)
            part = jnp.zeros((16,), jnp.int32)
            for ww in range(_NW - 1):
                row = grid_loc[ww, pl.ds(16 * i, 16)]
                part = part + row * (w > ww).astype(jnp.int32)
            off_loc[pl.ds(16 * i, 16)] = excl + part

    def radix_pass(shift, first, kout, vout, keys_out=True):
        zero_hist()

        def hist_body(i, c):
            if first:
                val_loc[pl.ds(16 * i, 16)] = base + 16 * i + lanes
            kv = key_loc[pl.ds(16 * i, 16)]
            d = lax.shift_right_logical(kv, shift) & 255
            cnt, last = plsc.scan_count(d)
            plsc.addupdate_scatter(hist_loc, [d], cnt, mask=last)
            return c

        lax.fori_loop(0, nv, hist_body, jnp.int32(0))
        pltpu.sync_copy(hist_loc, histg.at[w])
        plsc.subcore_barrier()
        compute_offsets()

        def perm_body(i, c):
            kv = key_loc[pl.ds(16 * i, 16)]
            d = lax.shift_right_logical(kv, shift) & 255
            cnt, last = plsc.scan_count(d)
            b_ = plsc.load_gather(off_loc, [d])
            pos_loc[pl.ds(16 * i, 16)] = b_ + cnt - 1
            plsc.addupdate_scatter(off_loc, [d], cnt, mask=last)
            return c

        lax.fori_loop(0, nv, perm_body, jnp.int32(0))

        def scat(sz):
            if keys_out:
                pltpu.async_copy(key_loc.at[pl.ds(0, sz)],
                                 kout.at[pos_loc.at[pl.ds(0, sz)]], sem).wait()
            pltpu.async_copy(val_loc.at[pl.ds(0, sz)],
                             vout.at[pos_loc.at[pl.ds(0, sz)]], sem).wait()

        @pl.when(not_last)
        def _():
            scat(_SHARD)

        @pl.when(is_last)
        def _():
            scat(_SHARD_LAST)

        plsc.subcore_barrier()

    def load_ping(kin, vin, with_vals):
        def ld(sz):
            pltpu.sync_copy(kin.at[pl.ds(base, sz)],
                            key_loc.at[pl.ds(0, sz)])
            if with_vals:
                pltpu.sync_copy(vin.at[pl.ds(base, sz)],
                                val_loc.at[pl.ds(0, sz)])

        @pl.when(not_last)
        def _():
            ld(_SHARD)

        @pl.when(is_last)
        def _():
            ld(_SHARD_LAST)

    load_ping(keys_hbm, None, False)
    radix_pass(0, True, kB, vB)
    load_ping(kB, vB, True)
    radix_pass(8, False, kA, vA)
    load_ping(kA, vA, True)
    radix_pass(16, False, kB, vB)
    load_ping(kB, vB, True)
    radix_pass(24, False, kA, vA, keys_out=False)

    def out_phase(sz, nxch, xrem):
        obase = w * _OSH
        pltpu.sync_copy(vA.at[pl.ds(_START + obase, sz)],
                        idx_loc.at[pl.ds(0, sz)])
        pltpu.sync_copy(idx_loc.at[pl.ds(0, sz)],
                        perm_hbm.at[pl.ds(obase, sz)])
        pltpu.async_copy(batch_hbm.at[idx_loc.at[pl.ds(0, sz)]],
                         btmp.at[pl.ds(0, sz)], sem).wait()
        pltpu.sync_copy(btmp.at[pl.ds(0, sz)],
                        batchc_hbm.at[pl.ds(obase, sz)])
        for src, dst in ((p0_hbm, p0c_hbm), (p1_hbm, p1c_hbm),
                         (p2_hbm, p2c_hbm)):
            pltpu.async_copy(src.at[idx_loc.at[pl.ds(0, sz)]],
                             ptmp.at[pl.ds(0, sz)], sem).wait()
            pltpu.sync_copy(ptmp.at[pl.ds(0, sz)],
                            dst.at[pl.ds(obase, sz)])
        for j in range(nxch):
            pltpu.async_copy(x_hbm.at[idx_loc.at[pl.ds(j * _XCH, _XCH)]],
                             xtmp, sem2).wait()
            pltpu.sync_copy(xtmp,
                            xsel_hbm.at[pl.ds(obase + j * _XCH, _XCH)])
        if xrem:
            pltpu.async_copy(x_hbm.at[idx_loc.at[pl.ds(nxch * _XCH, xrem)]],
                             xtmp.at[pl.ds(0, xrem)], sem2).wait()
            pltpu.sync_copy(xtmp.at[pl.ds(0, xrem)],
                            xsel_hbm.at[pl.ds(obase + nxch * _XCH, xrem)])

    @pl.when(not_last)
    def _():
        out_phase(_OSH, _OSH // _XCH, 0)

    @pl.when(is_last)
    def _():
        out_phase(_OSH_LAST, _OSH_LAST // _XCH, _OSH_LAST % _XCH)


def _sc_sort_gather(keys, p0, p1, p2, batch, x):
    mesh = plsc.VectorSubcoreMesh(
        core_axis_name="c", subcore_axis_name="s", num_cores=1)
    f = pl.kernel(
        _sc_body,
        out_type=[
            jax.ShapeDtypeStruct((_KEEP,), jnp.int32),
            jax.ShapeDtypeStruct((_KEEP,), jnp.float32),
            jax.ShapeDtypeStruct((_KEEP,), jnp.float32),
            jax.ShapeDtypeStruct((_KEEP,), jnp.float32),
            jax.ShapeDtypeStruct((_KEEP,), jnp.int32),
            jax.ShapeDtypeStruct((_KEEP, _D), jnp.float32),
        ],
        mesh=mesh,
        compiler_params=pltpu.CompilerParams(needs_layout_passes=False),
        scratch_types=[
            pltpu.VMEM_SHARED((_N + 96,), jnp.int32),
            pltpu.VMEM_SHARED((_N + 96,), jnp.int32),
            pltpu.VMEM_SHARED((_N + 96,), jnp.int32),
            pltpu.VMEM_SHARED((_N + 96,), jnp.int32),
            pltpu.VMEM_SHARED((_NW, 256), jnp.int32),
            pltpu.VMEM((_SHARD,), jnp.int32),
            pltpu.VMEM((_SHARD,), jnp.int32),
            pltpu.VMEM((_SHARD,), jnp.int32),
            pltpu.VMEM((_NW, 256), jnp.int32),
            pltpu.VMEM((256,), jnp.int32),
            pltpu.VMEM((256,), jnp.int32),
            pltpu.VMEM((_OSH,), jnp.int32),
            pltpu.VMEM((_XCH, _D), jnp.float32),
            pltpu.VMEM((_OSH,), jnp.float32),
            pltpu.VMEM((_OSH,), jnp.int32),
            pltpu.SemaphoreType.DMA,
            pltpu.SemaphoreType.DMA,
        ],
    )
    return f(keys, p0, p1, p2, batch, x)


_MBLK = 2048


def _mm_body(xs_ref, w_ref, b_ref, o_ref):
    o_ref[...] = (
        jnp.dot(xs_ref[...], w_ref[...], preferred_element_type=jnp.float32)
        + b_ref[...]
    )


def _matmul(x_sel, W, b):
    return pl.pallas_call(
        _mm_body,
        grid=(pl.cdiv(_KEEP, _MBLK),),
        in_specs=[
            pl.BlockSpec((_MBLK, _D), lambda i: (i, 0)),
            pl.BlockSpec((_D, _D), lambda i: (0, 0)),
            pl.BlockSpec((1, _D), lambda i: (0, 0)),
        ],
        out_specs=pl.BlockSpec((_MBLK, _D), lambda i: (i, 0)),
        out_shape=jax.ShapeDtypeStruct((_KEEP, _D), jnp.float32),
    )(x_sel, W, b.reshape(1, _D))


def kernel(x, pos, batch, W, b):
    scores = _scores(x)
    keys = lax.bitcast_convert_type(scores, jnp.int32)
    perm, p0c, p1c, p2c, batch_c, x_sel = _sc_sort_gather(
        keys, pos[:, 0], pos[:, 1], pos[:, 2], batch, x)
    x_c = _matmul(x_sel, W, b)
    pos_c = jnp.stack([p0c, p1c, p2c], axis=1)
    return (x_c, pos_c, batch_c)


# speedup vs baseline: 3.8271x; 1.0384x over previous
import jax
import jax.numpy as jnp
from jax import lax
from jax.experimental import pallas as pl
from jax.experimental.pallas import tpu as pltpu
from jax.experimental.pallas import tpu_sc as plsc

_N = 100000
_D = 128
_KEEP = 25000
_START = _N - _KEEP

_NW = 16
_SHARD = 6256
_SHARD_LAST = _N - 15 * _SHARD
_NV = _SHARD // 16
_NV_LAST = _SHARD_LAST // 16

_OSH = 1568
_OSH_LAST = _KEEP - 15 * _OSH
_XCH = 256


_BLK = 2048


def _scores_body(x_ref, o_ref):
    t = x_ref[...]
    t = (t * t).T
    u = t[0:8, :]
    for k in range(1, 16):
        u = u + t[8 * k:8 * k + 8, :]
    v = u[0:4, :] + u[4:8, :]
    w = v[0:2, :] + v[2:4, :]
    o_ref[...] = jnp.sqrt(w[0, :] + w[1, :])


def _scores(x):
    return pl.pallas_call(
        _scores_body,
        grid=(pl.cdiv(_N, _BLK),),
        in_specs=[pl.BlockSpec((_BLK, _D), lambda i: (i, 0))],
        out_specs=pl.BlockSpec((_BLK,), lambda i: (i,)),
        out_shape=jax.ShapeDtypeStruct((_N,), jnp.float32),
    )(x)


def _sc_body(keys_hbm, p0_hbm, p1_hbm, p2_hbm, batch_hbm, x_hbm,
             perm_hbm, p0c_hbm, p1c_hbm, p2c_hbm, batchc_hbm, xsel_hbm,
             kA, vA, kB, vB, histg,
             key_loc, val_loc, pos_loc, grid_loc, off_loc, hist_loc,
             idx_loc, xtmp0, xtmp1, ptmp0, ptmp1, ptmp2, btmp, sem, sem2):
    xtmps = (xtmp0, xtmp1)
    ptmps = (ptmp0, ptmp1, ptmp2)
    w = lax.axis_index("s")
    is_last = w == _NW - 1
    not_last = jnp.logical_not(is_last)
    base = w * _SHARD
    nv = jnp.where(is_last, _NV_LAST, _NV)
    lanes = lax.iota(jnp.int32, 16)

    def zero_hist():
        for i in range(16):
            hist_loc[pl.ds(16 * i, 16)] = jnp.zeros((16,), jnp.int32)

    def compute_offsets():
        pltpu.sync_copy(histg, grid_loc)
        carry = jnp.int32(0)
        for i in range(16):
            tot = jnp.zeros((16,), jnp.int32)
            for ww in range(_NW):
                tot = tot + grid_loc[ww, pl.ds(16 * i, 16)]
            csum = plsc.cumsum(tot)
            excl = (csum - tot) + carry
            carry = carry + jnp.sum(tot)
            part = jnp.zeros((16,), jnp.int32)
            for ww in range(_NW - 1):
                row = grid_loc[ww, pl.ds(16 * i, 16)]
                part = part + row * (w > ww).astype(jnp.int32)
            off_loc[pl.ds(16 * i, 16)] = excl + part

    def radix_pass(shift, first, kout, vout, keys_out=True):
        zero_hist()

        def hist_body(i, c):
            if first:
                val_loc[pl.ds(16 * i, 16)] = base + 16 * i + lanes
            kv = key_loc[pl.ds(16 * i, 16)]
            d = lax.shift_right_logical(kv, shift) & 255
            cnt, last = plsc.scan_count(d)
            plsc.addupdate_scatter(hist_loc, [d], cnt, mask=last)
            return c

        lax.fori_loop(0, nv, hist_body, jnp.int32(0))
        pltpu.sync_copy(hist_loc, histg.at[w])
        plsc.subcore_barrier()
        compute_offsets()

        def perm_body(i, c):
            kv = key_loc[pl.ds(16 * i, 16)]
            d = lax.shift_right_logical(kv, shift) & 255
            cnt, last = plsc.scan_count(d)
            b_ = plsc.load_gather(off_loc, [d])
            pos_loc[pl.ds(16 * i, 16)] = b_ + cnt - 1
            plsc.addupdate_scatter(off_loc, [d], cnt, mask=last)
            return c

        lax.fori_loop(0, nv, perm_body, jnp.int32(0))

        def scat(sz):
            cps = []
            if keys_out:
                cps.append(pltpu.async_copy(
                    key_loc.at[pl.ds(0, sz)],
                    kout.at[pos_loc.at[pl.ds(0, sz)]], sem))
            cps.append(pltpu.async_copy(
                val_loc.at[pl.ds(0, sz)],
                vout.at[pos_loc.at[pl.ds(0, sz)]], sem))
            for c in cps:
                c.wait()

        @pl.when(not_last)
        def _():
            scat(_SHARD)

        @pl.when(is_last)
        def _():
            scat(_SHARD_LAST)

        plsc.subcore_barrier()

    def load_ping(kin, vin, with_vals):
        def ld(sz):
            pltpu.sync_copy(kin.at[pl.ds(base, sz)],
                            key_loc.at[pl.ds(0, sz)])
            if with_vals:
                pltpu.sync_copy(vin.at[pl.ds(base, sz)],
                                val_loc.at[pl.ds(0, sz)])

        @pl.when(not_last)
        def _():
            ld(_SHARD)

        @pl.when(is_last)
        def _():
            ld(_SHARD_LAST)

    load_ping(keys_hbm, None, False)
    radix_pass(0, True, kB, vB)
    load_ping(kB, vB, True)
    radix_pass(8, False, kA, vA)
    load_ping(kA, vA, True)
    radix_pass(16, False, kB, vB)
    load_ping(kB, vB, True)
    radix_pass(24, False, kA, vA, keys_out=False)

    def out_phase(sz, nxch, xrem):
        obase = w * _OSH
        pltpu.sync_copy(vA.at[pl.ds(_START + obase, sz)],
                        idx_loc.at[pl.ds(0, sz)])
        pltpu.sync_copy(idx_loc.at[pl.ds(0, sz)],
                        perm_hbm.at[pl.ds(obase, sz)])
        cb = pltpu.async_copy(batch_hbm.at[idx_loc.at[pl.ds(0, sz)]],
                              btmp.at[pl.ds(0, sz)], sem)
        srcs = (p0_hbm, p1_hbm, p2_hbm)
        dsts = (p0c_hbm, p1c_hbm, p2c_hbm)
        cps = [pltpu.async_copy(srcs[c].at[idx_loc.at[pl.ds(0, sz)]],
                                ptmps[c].at[pl.ds(0, sz)], sem)
               for c in range(3)]
        cb.wait()
        pltpu.sync_copy(btmp.at[pl.ds(0, sz)],
                        batchc_hbm.at[pl.ds(obase, sz)])
        for c in range(3):
            cps[c].wait()
            pltpu.sync_copy(ptmps[c].at[pl.ds(0, sz)],
                            dsts[c].at[pl.ds(obase, sz)])

        def start(j, buf, n):
            return pltpu.async_copy(
                x_hbm.at[idx_loc.at[pl.ds(j * _XCH, n)]],
                xtmps[buf].at[pl.ds(0, n)], sem2)

        nch = nxch + (1 if xrem else 0)
        hnd = [None, None]
        hnd[0] = start(0, 0, _XCH if nxch else xrem)
        for j in range(1, nch):
            n = _XCH if j < nxch else xrem
            hnd[j % 2] = start(j, j % 2, n)
            np_ = _XCH if (j - 1) < nxch else xrem
            hnd[(j - 1) % 2].wait()
            pltpu.sync_copy(xtmps[(j - 1) % 2].at[pl.ds(0, np_)],
                            xsel_hbm.at[pl.ds(obase + (j - 1) * _XCH, np_)])
        nl = _XCH if (nch - 1) < nxch else xrem
        hnd[(nch - 1) % 2].wait()
        pltpu.sync_copy(xtmps[(nch - 1) % 2].at[pl.ds(0, nl)],
                        xsel_hbm.at[pl.ds(obase + (nch - 1) * _XCH, nl)])

    @pl.when(not_last)
    def _():
        out_phase(_OSH, _OSH // _XCH, _OSH % _XCH)

    @pl.when(is_last)
    def _():
        out_phase(_OSH_LAST, _OSH_LAST // _XCH, _OSH_LAST % _XCH)


def _sc_sort_gather(keys, p0, p1, p2, batch, x):
    mesh = plsc.VectorSubcoreMesh(
        core_axis_name="c", subcore_axis_name="s", num_cores=1)
    f = pl.kernel(
        _sc_body,
        out_type=[
            jax.ShapeDtypeStruct((_KEEP,), jnp.int32),
            jax.ShapeDtypeStruct((_KEEP,), jnp.float32),
            jax.ShapeDtypeStruct((_KEEP,), jnp.float32),
            jax.ShapeDtypeStruct((_KEEP,), jnp.float32),
            jax.ShapeDtypeStruct((_KEEP,), jnp.int32),
            jax.ShapeDtypeStruct((_KEEP, _D), jnp.float32),
        ],
        mesh=mesh,
        compiler_params=pltpu.CompilerParams(needs_layout_passes=False),
        scratch_types=[
            pltpu.VMEM_SHARED((_N + 96,), jnp.int32),
            pltpu.VMEM_SHARED((_N + 96,), jnp.int32),
            pltpu.VMEM_SHARED((_N + 96,), jnp.int32),
            pltpu.VMEM_SHARED((_N + 96,), jnp.int32),
            pltpu.VMEM_SHARED((_NW, 256), jnp.int32),
            pltpu.VMEM((_SHARD,), jnp.int32),
            pltpu.VMEM((_SHARD,), jnp.int32),
            pltpu.VMEM((_SHARD,), jnp.int32),
            pltpu.VMEM((_NW, 256), jnp.int32),
            pltpu.VMEM((256,), jnp.int32),
            pltpu.VMEM((256,), jnp.int32),
            pltpu.VMEM((_OSH,), jnp.int32),
            pltpu.VMEM((_XCH, _D), jnp.float32),
            pltpu.VMEM((_XCH, _D), jnp.float32),
            pltpu.VMEM((_OSH,), jnp.float32),
            pltpu.VMEM((_OSH,), jnp.float32),
            pltpu.VMEM((_OSH,), jnp.float32),
            pltpu.VMEM((_OSH,), jnp.int32),
            pltpu.SemaphoreType.DMA,
            pltpu.SemaphoreType.DMA,
        ],
    )
    return f(keys, p0, p1, p2, batch, x)


_MBLK = 2048


def _mm_body(xs_ref, w_ref, b_ref, o_ref):
    o_ref[...] = (
        jnp.dot(xs_ref[...], w_ref[...], preferred_element_type=jnp.float32)
        + b_ref[...]
    )


def _matmul(x_sel, W, b):
    return pl.pallas_call(
        _mm_body,
        grid=(pl.cdiv(_KEEP, _MBLK),),
        in_specs=[
            pl.BlockSpec((_MBLK, _D), lambda i: (i, 0)),
            pl.BlockSpec((_D, _D), lambda i: (0, 0)),
            pl.BlockSpec((1, _D), lambda i: (0, 0)),
        ],
        out_specs=pl.BlockSpec((_MBLK, _D), lambda i: (i, 0)),
        out_shape=jax.ShapeDtypeStruct((_KEEP, _D), jnp.float32),
    )(x_sel, W, b.reshape(1, _D))


def kernel(x, pos, batch, W, b):
    scores = _scores(x)
    keys = lax.bitcast_convert_type(scores, jnp.int32)
    perm, p0c, p1c, p2c, batch_c, x_sel = _sc_sort_gather(
        keys, pos[:, 0], pos[:, 1], pos[:, 2], batch, x)
    x_c = _matmul(x_sel, W, b)
    pos_c = jnp.stack([p0c, p1c, p2c], axis=1)
    return (x_c, pos_c, batch_c)


# speedup vs baseline: 3.8675x; 1.0106x over previous
import jax
import jax.numpy as jnp
from jax import lax
from jax.experimental import pallas as pl
from jax.experimental.pallas import tpu as pltpu
from jax.experimental.pallas import tpu_sc as plsc

_N = 100000
_D = 128
_KEEP = 25000
_START = _N - _KEEP

_NW = 16
_SHARD = 6256
_SHARD_LAST = _N - 15 * _SHARD
_NV = _SHARD // 16
_NV_LAST = _SHARD_LAST // 16

_OSH = 1568
_OSH_LAST = _KEEP - 15 * _OSH
_XCH = 256


_BLK = 2048


def _scores_body(x_ref, o_ref):
    t = x_ref[...]
    t = (t * t).T
    u = t[0:8, :]
    for k in range(1, 16):
        u = u + t[8 * k:8 * k + 8, :]
    v = u[0:4, :] + u[4:8, :]
    w = v[0:2, :] + v[2:4, :]
    s = jnp.sqrt(w[0, :] + w[1, :])
    o_ref[...] = lax.bitcast_convert_type(s, jnp.int32)


def _scores(x):
    return pl.pallas_call(
        _scores_body,
        grid=(pl.cdiv(_N, _BLK),),
        in_specs=[pl.BlockSpec((_BLK, _D), lambda i: (i, 0))],
        out_specs=pl.BlockSpec((_BLK,), lambda i: (i,)),
        out_shape=jax.ShapeDtypeStruct((_N,), jnp.int32),
    )(x)


def _sc_body(keys_hbm, p0_hbm, p1_hbm, p2_hbm, batch_hbm, x_hbm,
             perm_hbm, p0c_hbm, p1c_hbm, p2c_hbm, batchc_hbm, xsel_hbm,
             kA, vA, kB, vB, histg,
             key_loc, val_loc, pos_loc, grid_loc, off_loc, hist_loc,
             idx_loc, xtmp0, xtmp1, ptmp0, ptmp1, ptmp2, btmp, sem, sem2):
    xtmps = (xtmp0, xtmp1)
    ptmps = (ptmp0, ptmp1, ptmp2)
    w = lax.axis_index("s")
    is_last = w == _NW - 1
    not_last = jnp.logical_not(is_last)
    base = w * _SHARD
    nv = jnp.where(is_last, _NV_LAST, _NV)
    lanes = lax.iota(jnp.int32, 16)

    def zero_hist():
        for i in range(16):
            hist_loc[pl.ds(16 * i, 16)] = jnp.zeros((16,), jnp.int32)

    def compute_offsets():
        pltpu.sync_copy(histg, grid_loc)
        carry = jnp.int32(0)
        for i in range(16):
            tot = jnp.zeros((16,), jnp.int32)
            for ww in range(_NW):
                tot = tot + grid_loc[ww, pl.ds(16 * i, 16)]
            csum = plsc.cumsum(tot)
            excl = (csum - tot) + carry
            carry = carry + jnp.sum(tot)
            part = jnp.zeros((16,), jnp.int32)
            for ww in range(_NW - 1):
                row = grid_loc[ww, pl.ds(16 * i, 16)]
                part = part + row * (w > ww).astype(jnp.int32)
            off_loc[pl.ds(16 * i, 16)] = excl + part

    def radix_pass(shift, first, kout, vout, keys_out=True):
        zero_hist()

        def hist_body(i, c):
            if first:
                val_loc[pl.ds(16 * i, 16)] = base + 16 * i + lanes
            kv = key_loc[pl.ds(16 * i, 16)]
            d = lax.shift_right_logical(kv, shift) & 255
            cnt, last = plsc.scan_count(d)
            plsc.addupdate_scatter(hist_loc, [d], cnt, mask=last)
            return c

        lax.fori_loop(0, nv, hist_body, jnp.int32(0))
        pltpu.sync_copy(hist_loc, histg.at[w])
        plsc.subcore_barrier()
        compute_offsets()

        def perm_body(i, c):
            kv = key_loc[pl.ds(16 * i, 16)]
            d = lax.shift_right_logical(kv, shift) & 255
            cnt, last = plsc.scan_count(d)
            b_ = plsc.load_gather(off_loc, [d])
            pos_loc[pl.ds(16 * i, 16)] = b_ + cnt - 1
            plsc.addupdate_scatter(off_loc, [d], cnt, mask=last)
            return c

        lax.fori_loop(0, nv, perm_body, jnp.int32(0))

        def scat(sz):
            cps = []
            if keys_out:
                cps.append(pltpu.async_copy(
                    key_loc.at[pl.ds(0, sz)],
                    kout.at[pos_loc.at[pl.ds(0, sz)]], sem))
            cps.append(pltpu.async_copy(
                val_loc.at[pl.ds(0, sz)],
                vout.at[pos_loc.at[pl.ds(0, sz)]], sem))
            for c in cps:
                c.wait()

        @pl.when(not_last)
        def _():
            scat(_SHARD)

        @pl.when(is_last)
        def _():
            scat(_SHARD_LAST)

        plsc.subcore_barrier()

    def load_ping(kin, vin, with_vals):
        def ld(sz):
            pltpu.sync_copy(kin.at[pl.ds(base, sz)],
                            key_loc.at[pl.ds(0, sz)])
            if with_vals:
                pltpu.sync_copy(vin.at[pl.ds(base, sz)],
                                val_loc.at[pl.ds(0, sz)])

        @pl.when(not_last)
        def _():
            ld(_SHARD)

        @pl.when(is_last)
        def _():
            ld(_SHARD_LAST)

    load_ping(keys_hbm, None, False)
    radix_pass(0, True, kB, vB)
    load_ping(kB, vB, True)
    radix_pass(8, False, kA, vA)
    load_ping(kA, vA, True)
    radix_pass(16, False, kB, vB)
    load_ping(kB, vB, True)
    radix_pass(24, False, kA, vA, keys_out=False)

    def out_phase(sz, nxch, xrem):
        obase = w * _OSH
        pltpu.sync_copy(vA.at[pl.ds(_START + obase, sz)],
                        idx_loc.at[pl.ds(0, sz)])
        pltpu.sync_copy(idx_loc.at[pl.ds(0, sz)],
                        perm_hbm.at[pl.ds(obase, sz)])
        cb = pltpu.async_copy(batch_hbm.at[idx_loc.at[pl.ds(0, sz)]],
                              btmp.at[pl.ds(0, sz)], sem)
        srcs = (p0_hbm, p1_hbm, p2_hbm)
        dsts = (p0c_hbm, p1c_hbm, p2c_hbm)
        cps = [pltpu.async_copy(srcs[c].at[idx_loc.at[pl.ds(0, sz)]],
                                ptmps[c].at[pl.ds(0, sz)], sem)
               for c in range(3)]
        cb.wait()
        pltpu.sync_copy(btmp.at[pl.ds(0, sz)],
                        batchc_hbm.at[pl.ds(obase, sz)])
        for c in range(3):
            cps[c].wait()
            pltpu.sync_copy(ptmps[c].at[pl.ds(0, sz)],
                            dsts[c].at[pl.ds(obase, sz)])

        def start(j, buf, n):
            return pltpu.async_copy(
                x_hbm.at[idx_loc.at[pl.ds(j * _XCH, n)]],
                xtmps[buf].at[pl.ds(0, n)], sem2)

        nch = nxch + (1 if xrem else 0)
        hnd = [None, None]
        hnd[0] = start(0, 0, _XCH if nxch else xrem)
        for j in range(1, nch):
            n = _XCH if j < nxch else xrem
            hnd[j % 2] = start(j, j % 2, n)
            np_ = _XCH if (j - 1) < nxch else xrem
            hnd[(j - 1) % 2].wait()
            pltpu.sync_copy(xtmps[(j - 1) % 2].at[pl.ds(0, np_)],
                            xsel_hbm.at[pl.ds(obase + (j - 1) * _XCH, np_)])
        nl = _XCH if (nch - 1) < nxch else xrem
        hnd[(nch - 1) % 2].wait()
        pltpu.sync_copy(xtmps[(nch - 1) % 2].at[pl.ds(0, nl)],
                        xsel_hbm.at[pl.ds(obase + (nch - 1) * _XCH, nl)])

    @pl.when(not_last)
    def _():
        out_phase(_OSH, _OSH // _XCH, _OSH % _XCH)

    @pl.when(is_last)
    def _():
        out_phase(_OSH_LAST, _OSH_LAST // _XCH, _OSH_LAST % _XCH)


def _sc_sort_gather(keys, p0, p1, p2, batch, x):
    mesh = plsc.VectorSubcoreMesh(
        core_axis_name="c", subcore_axis_name="s", num_cores=1)
    f = pl.kernel(
        _sc_body,
        out_type=[
            jax.ShapeDtypeStruct((_KEEP,), jnp.int32),
            jax.ShapeDtypeStruct((_KEEP,), jnp.float32),
            jax.ShapeDtypeStruct((_KEEP,), jnp.float32),
            jax.ShapeDtypeStruct((_KEEP,), jnp.float32),
            jax.ShapeDtypeStruct((_KEEP,), jnp.int32),
            jax.ShapeDtypeStruct((_KEEP, _D), jnp.float32),
        ],
        mesh=mesh,
        compiler_params=pltpu.CompilerParams(needs_layout_passes=False),
        scratch_types=[
            pltpu.VMEM_SHARED((_N + 96,), jnp.int32),
            pltpu.VMEM_SHARED((_N + 96,), jnp.int32),
            pltpu.VMEM_SHARED((_N + 96,), jnp.int32),
            pltpu.VMEM_SHARED((_N + 96,), jnp.int32),
            pltpu.VMEM_SHARED((_NW, 256), jnp.int32),
            pltpu.VMEM((_SHARD,), jnp.int32),
            pltpu.VMEM((_SHARD,), jnp.int32),
            pltpu.VMEM((_SHARD,), jnp.int32),
            pltpu.VMEM((_NW, 256), jnp.int32),
            pltpu.VMEM((256,), jnp.int32),
            pltpu.VMEM((256,), jnp.int32),
            pltpu.VMEM((_OSH,), jnp.int32),
            pltpu.VMEM((_XCH, _D), jnp.float32),
            pltpu.VMEM((_XCH, _D), jnp.float32),
            pltpu.VMEM((_OSH,), jnp.float32),
            pltpu.VMEM((_OSH,), jnp.float32),
            pltpu.VMEM((_OSH,), jnp.float32),
            pltpu.VMEM((_OSH,), jnp.int32),
            pltpu.SemaphoreType.DMA,
            pltpu.SemaphoreType.DMA,
        ],
    )
    return f(keys, p0, p1, p2, batch, x)


_MBLK = 2048


def _mm_body(xs_ref, w_ref, b_ref, o_ref):
    o_ref[...] = (
        jnp.dot(xs_ref[...], w_ref[...], preferred_element_type=jnp.float32)
        + b_ref[...]
    )


def _matmul(x_sel, W, b):
    return pl.pallas_call(
        _mm_body,
        grid=(pl.cdiv(_KEEP, _MBLK),),
        in_specs=[
            pl.BlockSpec((_MBLK, _D), lambda i: (i, 0)),
            pl.BlockSpec((_D, _D), lambda i: (0, 0)),
            pl.BlockSpec((1, _D), lambda i: (0, 0)),
        ],
        out_specs=pl.BlockSpec((_MBLK, _D), lambda i: (i, 0)),
        out_shape=jax.ShapeDtypeStruct((_KEEP, _D), jnp.float32),
    )(x_sel, W, b.reshape(1, _D))


def kernel(x, pos, batch, W, b):
    keys = _scores(x)
    perm, p0c, p1c, p2c, batch_c, x_sel = _sc_sort_gather(
        keys, pos[:, 0], pos[:, 1], pos[:, 2], batch, x)
    x_c = _matmul(x_sel, W, b)
    pos_c = jnp.stack([p0c, p1c, p2c], axis=1)
    return (x_c, pos_c, batch_c)
